# baseline (speedup 1.0000x reference)
import functools

import jax
import jax.numpy as jnp
from jax import lax
from jax.experimental import pallas as pl
from jax.experimental.pallas import tpu as pltpu
from jax.experimental.pallas import tpu_sc as plsc

FN, FT = 48, 256
AN, AT = 96, 96
D = 256
H = 256
NB = 16

_INTERPRET = False


def _sc_gather(emb, idx_f, idx_a):
    NF = FN * FT
    NA = AN * AT
    info = plsc.get_sparse_core_info()
    NC, NS = info.num_cores, info.num_subcores
    NW = NC * NS
    bf = NF // NW
    ba = NA // NW
    mesh = plsc.VectorSubcoreMesh(core_axis_name="c", subcore_axis_name="s")

    @functools.partial(
        pl.kernel, mesh=mesh,
        out_type=(jax.ShapeDtypeStruct((NF, D), jnp.float32),
                  jax.ShapeDtypeStruct((NA, D), jnp.float32)),
        scratch_types=[
            pltpu.VMEM((bf,), jnp.int32),
            pltpu.VMEM((ba,), jnp.int32),
            pltpu.VMEM((bf, D), jnp.float32),
            pltpu.SemaphoreType.DMA,
        ],
    )
    def k(emb_hbm, idxf_hbm, idxa_hbm, outf_hbm, outa_hbm,
          idxf_v, idxa_v, rows_v, sem):
        wid = lax.axis_index("s") * NC + lax.axis_index("c")
        base_f = wid * bf
        pltpu.sync_copy(idxf_hbm.at[pl.ds(base_f, bf)], idxf_v)
        pltpu.async_copy(emb_hbm.at[idxf_v], rows_v, sem).wait()
        pltpu.sync_copy(rows_v, outf_hbm.at[pl.ds(base_f, bf)])
        base_a = wid * ba
        pltpu.sync_copy(idxa_hbm.at[pl.ds(base_a, ba)], idxa_v)
        pltpu.async_copy(emb_hbm.at[idxa_v], rows_v.at[pl.ds(0, ba)], sem).wait()
        pltpu.sync_copy(rows_v.at[pl.ds(0, ba)], outa_hbm.at[pl.ds(base_a, ba)])

    return k(emb, idx_f, idx_a)


def _sigm(x):
    return 0.5 * (jnp.tanh(0.5 * x) + 1.0)


def _lstm_step(x, h, c, acc, Wih, Whh, b, m, mask_state):
    g = (jnp.dot(x, Wih, preferred_element_type=jnp.float32)
         + jnp.dot(h, Whh, preferred_element_type=jnp.float32) + b)
    i = _sigm(g[:, 0:H])
    f = _sigm(g[:, H:2 * H])
    gg = jnp.tanh(g[:, 2 * H:3 * H])
    o = _sigm(g[:, 3 * H:4 * H])
    c_new = f * c + i * gg
    h_new = o * jnp.tanh(c_new)
    if mask_state:
        h2 = jnp.where(m, h_new, h.astype(jnp.float32)).astype(jnp.bfloat16)
        c2 = jnp.where(m, c_new, c)
    else:
        h2 = h_new.astype(jnp.bfloat16)
        c2 = c_new
    acc2 = acc + jnp.where(m, h_new, 0.0)
    return h2, c2, acc2


def _tc_body(ef_ref, ea_ref, lensf_ref, lensa_ref, fidx_ref, aidx_ref,
             WihFf_ref, WhhFf_ref, bFf_ref, WihFr_ref, WhhFr_ref, bFr_ref,
             WihAf_ref, WhhAf_ref, bAf_ref, WihAr_ref, WhhAr_ref, bAr_ref,
             W1_ref, b1_ref, W2_ref, b2_ref, out_ref):
    WihFf = WihFf_ref[...]
    WhhFf = WhhFf_ref[...]
    bFf = bFf_ref[...]
    WihFr = WihFr_ref[...]
    WhhFr = WhhFr_ref[...]
    bFr = bFr_ref[...]
    WihAf = WihAf_ref[...]
    WhhAf = WhhAf_ref[...]
    bAf = bAf_ref[...]
    WihAr = WihAr_ref[...]
    WhhAr = WhhAr_ref[...]
    bAr = bAr_ref[...]
    lens_f = lensf_ref[...]
    lens_a = lensa_ref[...]

    def facts_step(s, c):
        hf, cf, af, hr, cr, ar = c
        hf, cf, af = _lstm_step(ef_ref[s], hf, cf, af, WihFf, WhhFf, bFf,
                                s < lens_f, False)
        tr = FT - 1 - s
        hr, cr, ar = _lstm_step(ef_ref[tr], hr, cr, ar, WihFr, WhhFr, bFr,
                                tr < lens_f, True)
        return hf, cf, af, hr, cr, ar

    def artis_step(s, c):
        hf, cf, af, hr, cr, ar = c
        hf, cf, af = _lstm_step(ea_ref[s], hf, cf, af, WihAf, WhhAf, bAf,
                                s < lens_a, False)
        tr = AT - 1 - s
        hr, cr, ar = _lstm_step(ea_ref[tr], hr, cr, ar, WihAr, WhhAr, bAr,
                                tr < lens_a, True)
        return hf, cf, af, hr, cr, ar

    zf = jnp.zeros((FN, H), jnp.float32)
    za = jnp.zeros((AN, H), jnp.float32)
    zfh = jnp.zeros((FN, H), jnp.bfloat16)
    zah = jnp.zeros((AN, H), jnp.bfloat16)
    cf0 = (zfh, zf, zf, zfh, zf, zf)
    ca0 = (zah, za, za, zah, za, za)

    def both_step(s, c):
        return facts_step(s, c[0]), artis_step(s, c[1])

    cf1, ca1 = lax.fori_loop(0, AT, both_step, (cf0, ca0))
    cf2 = lax.fori_loop(AT, FT, facts_step, cf1)
    enc_f = jnp.concatenate([cf2[2], cf2[5]], axis=1)
    enc_a = jnp.concatenate([ca1[2], ca1[5]], axis=1)

    iota_f = lax.broadcasted_iota(jnp.int32, (NB, FN), 1)
    iota_a = lax.broadcasted_iota(jnp.int32, (NB, AN), 1)
    fidx = fidx_ref[...]
    aidx = aidx_ref[...]
    Pf = jnp.zeros((NB, FN), jnp.float32)
    for k in range(fidx.shape[1]):
        Pf = Pf + (iota_f == fidx[:, k:k + 1]).astype(jnp.float32)
    Pa = jnp.zeros((NB, AN), jnp.float32)
    for k in range(aidx.shape[1]):
        Pa = Pa + (iota_a == aidx[:, k:k + 1]).astype(jnp.float32)
    sf = jnp.dot(Pf, enc_f, preferred_element_type=jnp.float32)
    sa = jnp.dot(Pa, enc_a, preferred_element_type=jnp.float32)

    x1 = jnp.tanh(jnp.concatenate([sf, sa], axis=1))
    inter = jnp.dot(x1, W1_ref[...], preferred_element_type=jnp.float32) + b1_ref[...]
    out_ref[...] = (jnp.dot(jnp.tanh(inter), W2_ref[...],
                            preferred_element_type=jnp.float32) + b2_ref[...])


def _tc_forward(ef_tm, ea_tm, lens_f, lens_a, fidx, aidx, *ws):
    return pl.pallas_call(
        _tc_body,
        out_shape=jax.ShapeDtypeStruct((NB, 12), jnp.float32),
        interpret=_INTERPRET,
    )(ef_tm, ea_tm, lens_f, lens_a, fidx, aidx, *ws)


def kernel(facts, fact_lens, artis, arti_lens, fact_indices, arti_indices, emb,
           fWih_f, fWhh_f, fbih_f, fbhh_f, fWih_r, fWhh_r, fbih_r, fbhh_r,
           aWih_f, aWhh_f, abih_f, abhh_f, aWih_r, aWhh_r, abih_r, abhh_r,
           W1, b1, W2, b2):
    idx_f = facts.T.reshape(-1).astype(jnp.int32)
    idx_a = artis.T.reshape(-1).astype(jnp.int32)
    ef_flat, ea_flat = _sc_gather(emb, idx_f, idx_a)
    ef_tm = ef_flat.reshape(FT, FN, D).astype(jnp.bfloat16)
    ea_tm = ea_flat.reshape(AT, AN, D).astype(jnp.bfloat16)

    bh = jnp.bfloat16
    ws = (fWih_f.T.astype(bh), fWhh_f.T.astype(bh), (fbih_f + fbhh_f)[None, :],
          fWih_r.T.astype(bh), fWhh_r.T.astype(bh), (fbih_r + fbhh_r)[None, :],
          aWih_f.T.astype(bh), aWhh_f.T.astype(bh), (abih_f + abhh_f)[None, :],
          aWih_r.T.astype(bh), aWhh_r.T.astype(bh), (abih_r + abhh_r)[None, :],
          W1.T, b1[None, :], W2.T, b2[None, :])

    return _tc_forward(
        ef_tm, ea_tm,
        fact_lens.astype(jnp.int32).reshape(FN, 1),
        arti_lens.astype(jnp.int32).reshape(AN, 1),
        fact_indices.astype(jnp.int32), arti_indices.astype(jnp.int32),
        *ws)


# speedup vs baseline: 1.1334x; 1.1334x over previous
import functools

import jax
import jax.numpy as jnp
from jax import lax
from jax.experimental import pallas as pl
from jax.experimental.pallas import tpu as pltpu
from jax.experimental.pallas import tpu_sc as plsc

FN, FT = 48, 256
AN, AT = 96, 96
D = 256
H = 256
NB = 16

_INTERPRET = False


def _sc_gather(emb, idx_f, idx_a):
    NF = FN * FT
    NA = AN * AT
    info = plsc.get_sparse_core_info()
    NC, NS = info.num_cores, info.num_subcores
    NW = NC * NS
    bf = NF // NW
    ba = NA // NW
    mesh = plsc.VectorSubcoreMesh(core_axis_name="c", subcore_axis_name="s")

    @functools.partial(
        pl.kernel, mesh=mesh,
        out_type=(jax.ShapeDtypeStruct((NF, D), jnp.float32),
                  jax.ShapeDtypeStruct((NA, D), jnp.float32)),
        scratch_types=[
            pltpu.VMEM((bf,), jnp.int32),
            pltpu.VMEM((ba,), jnp.int32),
            pltpu.VMEM((bf, D), jnp.float32),
            pltpu.SemaphoreType.DMA,
        ],
    )
    def k(emb_hbm, idxf_hbm, idxa_hbm, outf_hbm, outa_hbm,
          idxf_v, idxa_v, rows_v, sem):
        wid = lax.axis_index("s") * NC + lax.axis_index("c")
        base_f = wid * bf
        pltpu.sync_copy(idxf_hbm.at[pl.ds(base_f, bf)], idxf_v)
        pltpu.async_copy(emb_hbm.at[idxf_v], rows_v, sem).wait()
        pltpu.sync_copy(rows_v, outf_hbm.at[pl.ds(base_f, bf)])
        base_a = wid * ba
        pltpu.sync_copy(idxa_hbm.at[pl.ds(base_a, ba)], idxa_v)
        pltpu.async_copy(emb_hbm.at[idxa_v], rows_v.at[pl.ds(0, ba)], sem).wait()
        pltpu.sync_copy(rows_v.at[pl.ds(0, ba)], outa_hbm.at[pl.ds(base_a, ba)])

    return k(emb, idx_f, idx_a)


def _sigm(x):
    return 0.5 * (jnp.tanh(0.5 * x) + 1.0)


def _lstm_step(x, h, c, acc, Wih, Whh, b, m, mask_state):
    g = (jnp.dot(x, Wih, preferred_element_type=jnp.float32)
         + jnp.dot(h, Whh, preferred_element_type=jnp.float32) + b)
    i = _sigm(g[:, 0:H])
    f = _sigm(g[:, H:2 * H])
    gg = jnp.tanh(g[:, 2 * H:3 * H])
    o = _sigm(g[:, 3 * H:4 * H])
    c_new = f * c + i * gg
    h_new = o * jnp.tanh(c_new)
    if mask_state:
        h2 = jnp.where(m, h_new, h.astype(jnp.float32)).astype(jnp.bfloat16)
        c2 = jnp.where(m, c_new, c)
    else:
        h2 = h_new.astype(jnp.bfloat16)
        c2 = c_new
    acc2 = acc + jnp.where(m, h_new, 0.0)
    return h2, c2, acc2


def _tc_body(ef_ref, ea_ref, lensf_ref, lensa_ref, fidx_ref, aidx_ref,
             WihFf_ref, WhhFf_ref, bFf_ref, WihFr_ref, WhhFr_ref, bFr_ref,
             WihAf_ref, WhhAf_ref, bAf_ref, WihAr_ref, WhhAr_ref, bAr_ref,
             W1_ref, b1_ref, W2_ref, b2_ref, out_ref):
    WihFf = WihFf_ref[...]
    WhhFf = WhhFf_ref[...]
    bFf = bFf_ref[...]
    WihFr = WihFr_ref[...]
    WhhFr = WhhFr_ref[...]
    bFr = bFr_ref[...]
    WihAf = WihAf_ref[...]
    WhhAf = WhhAf_ref[...]
    bAf = bAf_ref[...]
    WihAr = WihAr_ref[...]
    WhhAr = WhhAr_ref[...]
    bAr = bAr_ref[...]
    lens_f = lensf_ref[...]
    lens_a = lensa_ref[...]

    def facts_step(s, c):
        hf, cf, af, hr, cr, ar = c
        hf, cf, af = _lstm_step(ef_ref[s], hf, cf, af, WihFf, WhhFf, bFf,
                                s < lens_f, False)
        tr = FT - 1 - s
        hr, cr, ar = _lstm_step(ef_ref[tr], hr, cr, ar, WihFr, WhhFr, bFr,
                                tr < lens_f, True)
        return hf, cf, af, hr, cr, ar

    def artis_step(s, c):
        hf, cf, af, hr, cr, ar = c
        hf, cf, af = _lstm_step(ea_ref[s], hf, cf, af, WihAf, WhhAf, bAf,
                                s < lens_a, False)
        tr = AT - 1 - s
        hr, cr, ar = _lstm_step(ea_ref[tr], hr, cr, ar, WihAr, WhhAr, bAr,
                                tr < lens_a, True)
        return hf, cf, af, hr, cr, ar

    zf = jnp.zeros((FN, H), jnp.float32)
    za = jnp.zeros((AN, H), jnp.float32)
    zfh = jnp.zeros((FN, H), jnp.bfloat16)
    zah = jnp.zeros((AN, H), jnp.bfloat16)
    cf0 = (zfh, zf, zf, zfh, zf, zf)
    ca0 = (zah, za, za, zah, za, za)

    def both_step(s, c):
        return facts_step(s, c[0]), artis_step(s, c[1])

    cf1, ca1 = lax.fori_loop(0, AT, both_step, (cf0, ca0), unroll=2)
    cf2 = lax.fori_loop(AT, FT, facts_step, cf1, unroll=2)
    enc_f = jnp.concatenate([cf2[2], cf2[5]], axis=1)
    enc_a = jnp.concatenate([ca1[2], ca1[5]], axis=1)

    iota_f = lax.broadcasted_iota(jnp.int32, (NB, FN), 1)
    iota_a = lax.broadcasted_iota(jnp.int32, (NB, AN), 1)
    fidx = fidx_ref[...]
    aidx = aidx_ref[...]
    Pf = jnp.zeros((NB, FN), jnp.float32)
    for k in range(fidx.shape[1]):
        Pf = Pf + (iota_f == fidx[:, k:k + 1]).astype(jnp.float32)
    Pa = jnp.zeros((NB, AN), jnp.float32)
    for k in range(aidx.shape[1]):
        Pa = Pa + (iota_a == aidx[:, k:k + 1]).astype(jnp.float32)
    sf = jnp.dot(Pf, enc_f, preferred_element_type=jnp.float32)
    sa = jnp.dot(Pa, enc_a, preferred_element_type=jnp.float32)

    x1 = jnp.tanh(jnp.concatenate([sf, sa], axis=1))
    inter = jnp.dot(x1, W1_ref[...], preferred_element_type=jnp.float32) + b1_ref[...]
    out_ref[...] = (jnp.dot(jnp.tanh(inter), W2_ref[...],
                            preferred_element_type=jnp.float32) + b2_ref[...])


def _tc_forward(ef_tm, ea_tm, lens_f, lens_a, fidx, aidx, *ws):
    return pl.pallas_call(
        _tc_body,
        out_shape=jax.ShapeDtypeStruct((NB, 12), jnp.float32),
        interpret=_INTERPRET,
    )(ef_tm, ea_tm, lens_f, lens_a, fidx, aidx, *ws)


def kernel(facts, fact_lens, artis, arti_lens, fact_indices, arti_indices, emb,
           fWih_f, fWhh_f, fbih_f, fbhh_f, fWih_r, fWhh_r, fbih_r, fbhh_r,
           aWih_f, aWhh_f, abih_f, abhh_f, aWih_r, aWhh_r, abih_r, abhh_r,
           W1, b1, W2, b2):
    idx_f = facts.T.reshape(-1).astype(jnp.int32)
    idx_a = artis.T.reshape(-1).astype(jnp.int32)
    ef_flat, ea_flat = _sc_gather(emb, idx_f, idx_a)
    ef_tm = ef_flat.reshape(FT, FN, D).astype(jnp.bfloat16)
    ea_tm = ea_flat.reshape(AT, AN, D).astype(jnp.bfloat16)

    bh = jnp.bfloat16
    ws = (fWih_f.T.astype(bh), fWhh_f.T.astype(bh), (fbih_f + fbhh_f)[None, :],
          fWih_r.T.astype(bh), fWhh_r.T.astype(bh), (fbih_r + fbhh_r)[None, :],
          aWih_f.T.astype(bh), aWhh_f.T.astype(bh), (abih_f + abhh_f)[None, :],
          aWih_r.T.astype(bh), aWhh_r.T.astype(bh), (abih_r + abhh_r)[None, :],
          W1.T, b1[None, :], W2.T, b2[None, :])

    return _tc_forward(
        ef_tm, ea_tm,
        fact_lens.astype(jnp.int32).reshape(FN, 1),
        arti_lens.astype(jnp.int32).reshape(AN, 1),
        fact_indices.astype(jnp.int32), arti_indices.astype(jnp.int32),
        *ws)


# speedup vs baseline: 1.3506x; 1.1916x over previous
import functools

import jax
import jax.numpy as jnp
from jax import lax
from jax.experimental import pallas as pl
from jax.experimental.pallas import tpu as pltpu
from jax.experimental.pallas import tpu_sc as plsc

FN, FT = 48, 256
AN, AT = 96, 96
D = 256
H = 256
NB = 16

_INTERPRET = False


def _sc_gather(emb, idx_f, idx_a):
    NF = FN * FT
    NA = AN * AT
    info = plsc.get_sparse_core_info()
    NC, NS = info.num_cores, info.num_subcores
    NW = NC * NS
    bf = NF // NW
    ba = NA // NW
    mesh = plsc.VectorSubcoreMesh(core_axis_name="c", subcore_axis_name="s")

    @functools.partial(
        pl.kernel, mesh=mesh,
        out_type=(jax.ShapeDtypeStruct((NF, D), jnp.float32),
                  jax.ShapeDtypeStruct((NA, D), jnp.float32)),
        scratch_types=[
            pltpu.VMEM((bf,), jnp.int32),
            pltpu.VMEM((ba,), jnp.int32),
            pltpu.VMEM((bf, D), jnp.float32),
            pltpu.SemaphoreType.DMA,
        ],
    )
    def k(emb_hbm, idxf_hbm, idxa_hbm, outf_hbm, outa_hbm,
          idxf_v, idxa_v, rows_v, sem):
        wid = lax.axis_index("s") * NC + lax.axis_index("c")
        base_f = wid * bf
        pltpu.sync_copy(idxf_hbm.at[pl.ds(base_f, bf)], idxf_v)
        pltpu.async_copy(emb_hbm.at[idxf_v], rows_v, sem).wait()
        pltpu.sync_copy(rows_v, outf_hbm.at[pl.ds(base_f, bf)])
        base_a = wid * ba
        pltpu.sync_copy(idxa_hbm.at[pl.ds(base_a, ba)], idxa_v)
        pltpu.async_copy(emb_hbm.at[idxa_v], rows_v.at[pl.ds(0, ba)], sem).wait()
        pltpu.sync_copy(rows_v.at[pl.ds(0, ba)], outa_hbm.at[pl.ds(base_a, ba)])

    return k(emb, idx_f, idx_a)


def _sigm(x):
    return 0.5 * (jnp.tanh(0.5 * x) + 1.0)


def _lstm_step(xp, h, c, acc, Whh, m, mask_state):
    g = xp + jnp.dot(h, Whh, preferred_element_type=jnp.float32)
    i = _sigm(g[:, 0:H])
    f = _sigm(g[:, H:2 * H])
    gg = jnp.tanh(g[:, 2 * H:3 * H])
    o = _sigm(g[:, 3 * H:4 * H])
    c_new = f * c + i * gg
    h_new = o * jnp.tanh(c_new)
    if mask_state:
        h2 = jnp.where(m, h_new, h.astype(jnp.float32)).astype(jnp.bfloat16)
        c2 = jnp.where(m, c_new, c)
    else:
        h2 = h_new.astype(jnp.bfloat16)
        c2 = c_new
    acc2 = acc + jnp.where(m, h_new, 0.0)
    return h2, c2, acc2


CH = 16
FCH = FT // CH
ACH = AT // CH


def _tc_body(ef_ref, ea_ref, lensf_ref, lensa_ref, fidx_ref, aidx_ref,
             WihFf_ref, WhhFf_ref, bFf_ref, WihFr_ref, WhhFr_ref, bFr_ref,
             WihAf_ref, WhhAf_ref, bAf_ref, WihAr_ref, WhhAr_ref, bAr_ref,
             W1_ref, b1_ref, W2_ref, b2_ref, out_ref,
             XPFf_ref, XPFr_ref, XPAf_ref, XPAr_ref):
    WhhFf = WhhFf_ref[...]
    WhhFr = WhhFr_ref[...]
    WhhAf = WhhAf_ref[...]
    WhhAr = WhhAr_ref[...]
    lens_f = lensf_ref[...]
    lens_a = lensa_ref[...]

    def facts_chunk_precompute(c):
        xf = ef_ref[pl.ds(c * CH, CH)].reshape(CH * FN, D)
        XPFf_ref[...] = (jnp.dot(xf, WihFf_ref[...],
                                 preferred_element_type=jnp.float32)
                         + bFf_ref[...])
        xr = ef_ref[pl.ds(FT - CH - c * CH, CH)].reshape(CH * FN, D)
        XPFr_ref[...] = (jnp.dot(xr, WihFr_ref[...],
                                 preferred_element_type=jnp.float32)
                         + bFr_ref[...])

    def artis_chunk_precompute(c):
        xa = ea_ref[pl.ds(c * CH, CH)].reshape(CH * AN, D)
        XPAf_ref[...] = (jnp.dot(xa, WihAf_ref[...],
                                 preferred_element_type=jnp.float32)
                         + bAf_ref[...])
        xr = ea_ref[pl.ds(AT - CH - c * CH, CH)].reshape(CH * AN, D)
        XPAr_ref[...] = (jnp.dot(xr, WihAr_ref[...],
                                 preferred_element_type=jnp.float32)
                         + bAr_ref[...])

    def facts_step(c, j, st):
        hf, cf, af, hr, cr, ar = st
        s = c * CH + j
        xpf = XPFf_ref[pl.ds(j * FN, FN)]
        hf, cf, af = _lstm_step(xpf, hf, cf, af, WhhFf, s < lens_f, False)
        tr = FT - 1 - s
        xpr = XPFr_ref[pl.ds((CH - 1 - j) * FN, FN)]
        hr, cr, ar = _lstm_step(xpr, hr, cr, ar, WhhFr, tr < lens_f, True)
        return hf, cf, af, hr, cr, ar

    def artis_step(c, j, st):
        hf, cf, af, hr, cr, ar = st
        s = c * CH + j
        xpf = XPAf_ref[pl.ds(j * AN, AN)]
        hf, cf, af = _lstm_step(xpf, hf, cf, af, WhhAf, s < lens_a, False)
        tr = AT - 1 - s
        xpr = XPAr_ref[pl.ds((CH - 1 - j) * AN, AN)]
        hr, cr, ar = _lstm_step(xpr, hr, cr, ar, WhhAr, tr < lens_a, True)
        return hf, cf, af, hr, cr, ar

    zf = jnp.zeros((FN, H), jnp.float32)
    za = jnp.zeros((AN, H), jnp.float32)
    zfh = jnp.zeros((FN, H), jnp.bfloat16)
    zah = jnp.zeros((AN, H), jnp.bfloat16)
    cf0 = (zfh, zf, zf, zfh, zf, zf)
    ca0 = (zah, za, za, zah, za, za)

    def outer_both(c, carry):
        facts_chunk_precompute(c)
        artis_chunk_precompute(c)

        def inner(j, st):
            return facts_step(c, j, st[0]), artis_step(c, j, st[1])

        return lax.fori_loop(0, CH, inner, carry, unroll=2)

    def outer_facts(c, carry):
        facts_chunk_precompute(c)

        def inner(j, st):
            return facts_step(c, j, st[0]), st[1]

        return lax.fori_loop(0, CH, inner, carry, unroll=2)

    cf1, ca1 = lax.fori_loop(0, ACH, outer_both, (cf0, ca0))
    cf2, _ = lax.fori_loop(ACH, FCH, outer_facts, (cf1, ca1))
    enc_f = jnp.concatenate([cf2[2], cf2[5]], axis=1)
    enc_a = jnp.concatenate([ca1[2], ca1[5]], axis=1)

    iota_f = lax.broadcasted_iota(jnp.int32, (NB, FN), 1)
    iota_a = lax.broadcasted_iota(jnp.int32, (NB, AN), 1)
    fidx = fidx_ref[...]
    aidx = aidx_ref[...]
    Pf = jnp.zeros((NB, FN), jnp.float32)
    for k in range(fidx.shape[1]):
        Pf = Pf + (iota_f == fidx[:, k:k + 1]).astype(jnp.float32)
    Pa = jnp.zeros((NB, AN), jnp.float32)
    for k in range(aidx.shape[1]):
        Pa = Pa + (iota_a == aidx[:, k:k + 1]).astype(jnp.float32)
    sf = jnp.dot(Pf, enc_f, preferred_element_type=jnp.float32)
    sa = jnp.dot(Pa, enc_a, preferred_element_type=jnp.float32)

    x1 = jnp.tanh(jnp.concatenate([sf, sa], axis=1))
    inter = jnp.dot(x1, W1_ref[...], preferred_element_type=jnp.float32) + b1_ref[...]
    out_ref[...] = (jnp.dot(jnp.tanh(inter), W2_ref[...],
                            preferred_element_type=jnp.float32) + b2_ref[...])


def _tc_forward(ef_tm, ea_tm, lens_f, lens_a, fidx, aidx, *ws):
    return pl.pallas_call(
        _tc_body,
        out_shape=jax.ShapeDtypeStruct((NB, 12), jnp.float32),
        scratch_shapes=[
            pltpu.VMEM((CH * FN, 4 * H), jnp.float32),
            pltpu.VMEM((CH * FN, 4 * H), jnp.float32),
            pltpu.VMEM((CH * AN, 4 * H), jnp.float32),
            pltpu.VMEM((CH * AN, 4 * H), jnp.float32),
        ],
        interpret=_INTERPRET,
    )(ef_tm, ea_tm, lens_f, lens_a, fidx, aidx, *ws)


def kernel(facts, fact_lens, artis, arti_lens, fact_indices, arti_indices, emb,
           fWih_f, fWhh_f, fbih_f, fbhh_f, fWih_r, fWhh_r, fbih_r, fbhh_r,
           aWih_f, aWhh_f, abih_f, abhh_f, aWih_r, aWhh_r, abih_r, abhh_r,
           W1, b1, W2, b2):
    idx_f = facts.T.reshape(-1).astype(jnp.int32)
    idx_a = artis.T.reshape(-1).astype(jnp.int32)
    ef_flat, ea_flat = _sc_gather(emb, idx_f, idx_a)
    ef_tm = ef_flat.reshape(FT, FN, D).astype(jnp.bfloat16)
    ea_tm = ea_flat.reshape(AT, AN, D).astype(jnp.bfloat16)

    bh = jnp.bfloat16
    ws = (fWih_f.T.astype(bh), fWhh_f.T.astype(bh), (fbih_f + fbhh_f)[None, :],
          fWih_r.T.astype(bh), fWhh_r.T.astype(bh), (fbih_r + fbhh_r)[None, :],
          aWih_f.T.astype(bh), aWhh_f.T.astype(bh), (abih_f + abhh_f)[None, :],
          aWih_r.T.astype(bh), aWhh_r.T.astype(bh), (abih_r + abhh_r)[None, :],
          W1.T, b1[None, :], W2.T, b2[None, :])

    return _tc_forward(
        ef_tm, ea_tm,
        fact_lens.astype(jnp.int32).reshape(FN, 1),
        arti_lens.astype(jnp.int32).reshape(AN, 1),
        fact_indices.astype(jnp.int32), arti_indices.astype(jnp.int32),
        *ws)


# speedup vs baseline: 1.4387x; 1.0653x over previous
import functools

import jax
import jax.numpy as jnp
from jax import lax
from jax.experimental import pallas as pl
from jax.experimental.pallas import tpu as pltpu
from jax.experimental.pallas import tpu_sc as plsc

FN, FT = 48, 256
AN, AT = 96, 96
D = 256
H = 256
NB = 16

_INTERPRET = False


def _sc_gather(emb, idx_f, idx_a):
    NF = FN * FT
    NA = AN * AT
    info = plsc.get_sparse_core_info()
    NC, NS = info.num_cores, info.num_subcores
    NW = NC * NS
    bf = NF // NW
    ba = NA // NW
    mesh = plsc.VectorSubcoreMesh(core_axis_name="c", subcore_axis_name="s")

    @functools.partial(
        pl.kernel, mesh=mesh,
        out_type=(jax.ShapeDtypeStruct((NF, D), jnp.float32),
                  jax.ShapeDtypeStruct((NA, D), jnp.float32)),
        scratch_types=[
            pltpu.VMEM((bf,), jnp.int32),
            pltpu.VMEM((ba,), jnp.int32),
            pltpu.VMEM((bf, D), jnp.float32),
            pltpu.SemaphoreType.DMA,
        ],
    )
    def k(emb_hbm, idxf_hbm, idxa_hbm, outf_hbm, outa_hbm,
          idxf_v, idxa_v, rows_v, sem):
        wid = lax.axis_index("s") * NC + lax.axis_index("c")
        base_f = wid * bf
        pltpu.sync_copy(idxf_hbm.at[pl.ds(base_f, bf)], idxf_v)
        pltpu.async_copy(emb_hbm.at[idxf_v], rows_v, sem).wait()
        pltpu.sync_copy(rows_v, outf_hbm.at[pl.ds(base_f, bf)])
        base_a = wid * ba
        pltpu.sync_copy(idxa_hbm.at[pl.ds(base_a, ba)], idxa_v)
        pltpu.async_copy(emb_hbm.at[idxa_v], rows_v.at[pl.ds(0, ba)], sem).wait()
        pltpu.sync_copy(rows_v.at[pl.ds(0, ba)], outa_hbm.at[pl.ds(base_a, ba)])

    return k(emb, idx_f, idx_a)


def _sigm(x):
    return 0.5 * (jnp.tanh(0.5 * x) + 1.0)


def _lstm_step(xp, h, c, acc, Whh, m, mask_state):
    g = xp + jnp.dot(h, Whh, preferred_element_type=jnp.float32)
    i = _sigm(g[:, 0:H])
    f = _sigm(g[:, H:2 * H])
    gg = jnp.tanh(g[:, 2 * H:3 * H])
    o = _sigm(g[:, 3 * H:4 * H])
    c_new = f * c + i * gg
    h_new = o * jnp.tanh(c_new)
    if mask_state:
        h2 = jnp.where(m, h_new, h.astype(jnp.float32)).astype(jnp.bfloat16)
        c2 = jnp.where(m, c_new, c)
    else:
        h2 = h_new.astype(jnp.bfloat16)
        c2 = c_new
    acc2 = acc + jnp.where(m, h_new, 0.0)
    return h2, c2, acc2


CH = 16
FCH = FT // CH
ACH = AT // CH


def _tc_body(ef_ref, ea_ref, lensf_ref, lensa_ref, fidx_ref, aidx_ref,
             WihFf_ref, WhhFf_ref, bFf_ref, WihFr_ref, WhhFr_ref, bFr_ref,
             WihAf_ref, WhhAf_ref, bAf_ref, WihAr_ref, WhhAr_ref, bAr_ref,
             W1_ref, b1_ref, W2_ref, b2_ref, out_ref,
             XPFf_ref, XPFr_ref, XPAf_ref, XPAr_ref):
    WhhFf = WhhFf_ref[...]
    WhhFr = WhhFr_ref[...]
    WhhAf = WhhAf_ref[...]
    WhhAr = WhhAr_ref[...]
    lens_f = lensf_ref[...]
    lens_a = lensa_ref[...]

    def facts_chunk_precompute(c):
        xf = ef_ref[pl.ds(c * CH, CH)].reshape(CH * FN, D)
        XPFf_ref[...] = (jnp.dot(xf, WihFf_ref[...],
                                 preferred_element_type=jnp.float32)
                         + bFf_ref[...])
        xr = ef_ref[pl.ds(FT - CH - c * CH, CH)].reshape(CH * FN, D)
        XPFr_ref[...] = (jnp.dot(xr, WihFr_ref[...],
                                 preferred_element_type=jnp.float32)
                         + bFr_ref[...])

    def artis_chunk_precompute(c):
        xa = ea_ref[pl.ds(c * CH, CH)].reshape(CH * AN, D)
        XPAf_ref[...] = (jnp.dot(xa, WihAf_ref[...],
                                 preferred_element_type=jnp.float32)
                         + bAf_ref[...])
        xr = ea_ref[pl.ds(AT - CH - c * CH, CH)].reshape(CH * AN, D)
        XPAr_ref[...] = (jnp.dot(xr, WihAr_ref[...],
                                 preferred_element_type=jnp.float32)
                         + bAr_ref[...])

    def facts_step(c, j, st):
        hf, cf, af, hr, cr, ar = st
        s = c * CH + j
        xpf = XPFf_ref[pl.ds(j * FN, FN)]
        hf, cf, af = _lstm_step(xpf, hf, cf, af, WhhFf, s < lens_f, False)
        tr = FT - 1 - s
        xpr = XPFr_ref[pl.ds((CH - 1 - j) * FN, FN)]
        hr, cr, ar = _lstm_step(xpr, hr, cr, ar, WhhFr, tr < lens_f, True)
        return hf, cf, af, hr, cr, ar

    def artis_step(c, j, st):
        hf, cf, af, hr, cr, ar = st
        s = c * CH + j
        xpf = XPAf_ref[pl.ds(j * AN, AN)]
        hf, cf, af = _lstm_step(xpf, hf, cf, af, WhhAf, s < lens_a, False)
        tr = AT - 1 - s
        xpr = XPAr_ref[pl.ds((CH - 1 - j) * AN, AN)]
        hr, cr, ar = _lstm_step(xpr, hr, cr, ar, WhhAr, tr < lens_a, True)
        return hf, cf, af, hr, cr, ar

    zf = jnp.zeros((FN, H), jnp.float32)
    za = jnp.zeros((AN, H), jnp.float32)
    zfh = jnp.zeros((FN, H), jnp.bfloat16)
    zah = jnp.zeros((AN, H), jnp.bfloat16)
    cf0 = (zfh, zf, zf, zfh, zf, zf)
    ca0 = (zah, za, za, zah, za, za)

    def outer_both(c, carry):
        facts_chunk_precompute(c)
        artis_chunk_precompute(c)

        def inner(j, st):
            return facts_step(c, j, st[0]), artis_step(c, j, st[1])

        return lax.fori_loop(0, CH, inner, carry, unroll=4)

    def outer_facts(c, carry):
        facts_chunk_precompute(c)

        def inner(j, st):
            return facts_step(c, j, st[0]), st[1]

        return lax.fori_loop(0, CH, inner, carry, unroll=4)

    cf1, ca1 = lax.fori_loop(0, ACH, outer_both, (cf0, ca0))
    cf2, _ = lax.fori_loop(ACH, FCH, outer_facts, (cf1, ca1))
    enc_f = jnp.concatenate([cf2[2], cf2[5]], axis=1)
    enc_a = jnp.concatenate([ca1[2], ca1[5]], axis=1)

    iota_f = lax.broadcasted_iota(jnp.int32, (NB, FN), 1)
    iota_a = lax.broadcasted_iota(jnp.int32, (NB, AN), 1)
    fidx = fidx_ref[...]
    aidx = aidx_ref[...]
    Pf = jnp.zeros((NB, FN), jnp.float32)
    for k in range(fidx.shape[1]):
        Pf = Pf + (iota_f == fidx[:, k:k + 1]).astype(jnp.float32)
    Pa = jnp.zeros((NB, AN), jnp.float32)
    for k in range(aidx.shape[1]):
        Pa = Pa + (iota_a == aidx[:, k:k + 1]).astype(jnp.float32)
    sf = jnp.dot(Pf, enc_f, preferred_element_type=jnp.float32)
    sa = jnp.dot(Pa, enc_a, preferred_element_type=jnp.float32)

    x1 = jnp.tanh(jnp.concatenate([sf, sa], axis=1))
    inter = jnp.dot(x1, W1_ref[...], preferred_element_type=jnp.float32) + b1_ref[...]
    out_ref[...] = (jnp.dot(jnp.tanh(inter), W2_ref[...],
                            preferred_element_type=jnp.float32) + b2_ref[...])


def _tc_forward(ef_tm, ea_tm, lens_f, lens_a, fidx, aidx, *ws):
    return pl.pallas_call(
        _tc_body,
        out_shape=jax.ShapeDtypeStruct((NB, 12), jnp.float32),
        scratch_shapes=[
            pltpu.VMEM((CH * FN, 4 * H), jnp.float32),
            pltpu.VMEM((CH * FN, 4 * H), jnp.float32),
            pltpu.VMEM((CH * AN, 4 * H), jnp.float32),
            pltpu.VMEM((CH * AN, 4 * H), jnp.float32),
        ],
        interpret=_INTERPRET,
    )(ef_tm, ea_tm, lens_f, lens_a, fidx, aidx, *ws)


def kernel(facts, fact_lens, artis, arti_lens, fact_indices, arti_indices, emb,
           fWih_f, fWhh_f, fbih_f, fbhh_f, fWih_r, fWhh_r, fbih_r, fbhh_r,
           aWih_f, aWhh_f, abih_f, abhh_f, aWih_r, aWhh_r, abih_r, abhh_r,
           W1, b1, W2, b2):
    idx_f = facts.T.reshape(-1).astype(jnp.int32)
    idx_a = artis.T.reshape(-1).astype(jnp.int32)
    ef_flat, ea_flat = _sc_gather(emb, idx_f, idx_a)
    ef_tm = ef_flat.reshape(FT, FN, D).astype(jnp.bfloat16)
    ea_tm = ea_flat.reshape(AT, AN, D).astype(jnp.bfloat16)

    bh = jnp.bfloat16
    ws = (fWih_f.T.astype(bh), fWhh_f.T.astype(bh), (fbih_f + fbhh_f)[None, :],
          fWih_r.T.astype(bh), fWhh_r.T.astype(bh), (fbih_r + fbhh_r)[None, :],
          aWih_f.T.astype(bh), aWhh_f.T.astype(bh), (abih_f + abhh_f)[None, :],
          aWih_r.T.astype(bh), aWhh_r.T.astype(bh), (abih_r + abhh_r)[None, :],
          W1.T, b1[None, :], W2.T, b2[None, :])

    return _tc_forward(
        ef_tm, ea_tm,
        fact_lens.astype(jnp.int32).reshape(FN, 1),
        arti_lens.astype(jnp.int32).reshape(AN, 1),
        fact_indices.astype(jnp.int32), arti_indices.astype(jnp.int32),
        *ws)


# speedup vs baseline: 1.4545x; 1.0109x over previous
import functools

import jax
import jax.numpy as jnp
from jax import lax
from jax.experimental import pallas as pl
from jax.experimental.pallas import tpu as pltpu
from jax.experimental.pallas import tpu_sc as plsc

FN, FT = 48, 256
AN, AT = 96, 96
D = 256
H = 256
NB = 16

_INTERPRET = False


def _sc_gather(emb, idx_f, idx_a):
    NF = FN * FT
    NA = AN * AT
    info = plsc.get_sparse_core_info()
    NC, NS = info.num_cores, info.num_subcores
    NW = NC * NS
    bf = NF // NW
    ba = NA // NW
    mesh = plsc.VectorSubcoreMesh(core_axis_name="c", subcore_axis_name="s")

    @functools.partial(
        pl.kernel, mesh=mesh,
        out_type=(jax.ShapeDtypeStruct((NF, D), jnp.float32),
                  jax.ShapeDtypeStruct((NA, D), jnp.float32)),
        scratch_types=[
            pltpu.VMEM((bf,), jnp.int32),
            pltpu.VMEM((ba,), jnp.int32),
            pltpu.VMEM((bf, D), jnp.float32),
            pltpu.SemaphoreType.DMA,
        ],
    )
    def k(emb_hbm, idxf_hbm, idxa_hbm, outf_hbm, outa_hbm,
          idxf_v, idxa_v, rows_v, sem):
        wid = lax.axis_index("s") * NC + lax.axis_index("c")
        base_f = wid * bf
        pltpu.sync_copy(idxf_hbm.at[pl.ds(base_f, bf)], idxf_v)
        pltpu.async_copy(emb_hbm.at[idxf_v], rows_v, sem).wait()
        pltpu.sync_copy(rows_v, outf_hbm.at[pl.ds(base_f, bf)])
        base_a = wid * ba
        pltpu.sync_copy(idxa_hbm.at[pl.ds(base_a, ba)], idxa_v)
        pltpu.async_copy(emb_hbm.at[idxa_v], rows_v.at[pl.ds(0, ba)], sem).wait()
        pltpu.sync_copy(rows_v.at[pl.ds(0, ba)], outa_hbm.at[pl.ds(base_a, ba)])

    return k(emb, idx_f, idx_a)


def _sigm(x):
    return 0.5 * (jnp.tanh(0.5 * x) + 1.0)


def _lstm_step(xp, h, c, acc, Whh, m, mask_state):
    g = xp + jnp.dot(h, Whh, preferred_element_type=jnp.float32)
    i = _sigm(g[:, 0:H])
    f = _sigm(g[:, H:2 * H])
    gg = jnp.tanh(g[:, 2 * H:3 * H])
    o = _sigm(g[:, 3 * H:4 * H])
    c_new = f * c + i * gg
    h_new = o * jnp.tanh(c_new)
    if mask_state:
        h2 = jnp.where(m, h_new, h.astype(jnp.float32)).astype(jnp.bfloat16)
        c2 = jnp.where(m, c_new, c)
    else:
        h2 = h_new.astype(jnp.bfloat16)
        c2 = c_new
    acc2 = acc + jnp.where(m, h_new, 0.0)
    return h2, c2, acc2


CH = 8
FCH = FT // CH
ACH = AT // CH


def _tc_body(ef_ref, ea_ref, lensf_ref, lensa_ref, fidx_ref, aidx_ref,
             WihFf_ref, WhhFf_ref, bFf_ref, WihFr_ref, WhhFr_ref, bFr_ref,
             WihAf_ref, WhhAf_ref, bAf_ref, WihAr_ref, WhhAr_ref, bAr_ref,
             W1_ref, b1_ref, W2_ref, b2_ref, out_ref,
             XPFfA_ref, XPFfB_ref, XPFrA_ref, XPFrB_ref,
             XPAfA_ref, XPAfB_ref, XPArA_ref, XPArB_ref):
    WhhFf = WhhFf_ref[...]
    WhhFr = WhhFr_ref[...]
    WhhAf = WhhAf_ref[...]
    WhhAr = WhhAr_ref[...]
    lens_f = lensf_ref[...]
    lens_a = lensa_ref[...]

    def pf(c, dstf, dstr):
        xf = ef_ref[pl.ds(c * CH, CH)].reshape(CH * FN, D)
        dstf[...] = (jnp.dot(xf, WihFf_ref[...],
                             preferred_element_type=jnp.float32) + bFf_ref[...])
        xr = ef_ref[pl.ds(FT - CH - c * CH, CH)].reshape(CH * FN, D)
        dstr[...] = (jnp.dot(xr, WihFr_ref[...],
                             preferred_element_type=jnp.float32) + bFr_ref[...])

    def pa(c, dstf, dstr):
        xa = ea_ref[pl.ds(c * CH, CH)].reshape(CH * AN, D)
        dstf[...] = (jnp.dot(xa, WihAf_ref[...],
                             preferred_element_type=jnp.float32) + bAf_ref[...])
        xr = ea_ref[pl.ds(AT - CH - c * CH, CH)].reshape(CH * AN, D)
        dstr[...] = (jnp.dot(xr, WihAr_ref[...],
                             preferred_element_type=jnp.float32) + bAr_ref[...])

    def fsteps(c, st, Ff_ref, Fr_ref):
        hf, cf, af, hr, cr, ar = st
        for j in range(CH):
            s = c * CH + j
            xpf = Ff_ref[pl.ds(j * FN, FN)]
            hf, cf, af = _lstm_step(xpf, hf, cf, af, WhhFf, s < lens_f, False)
            tr = FT - 1 - s
            xpr = Fr_ref[pl.ds((CH - 1 - j) * FN, FN)]
            hr, cr, ar = _lstm_step(xpr, hr, cr, ar, WhhFr, tr < lens_f, True)
        return hf, cf, af, hr, cr, ar

    def asteps(c, st, Af_ref, Ar_ref):
        hf, cf, af, hr, cr, ar = st
        for j in range(CH):
            s = c * CH + j
            xpf = Af_ref[pl.ds(j * AN, AN)]
            hf, cf, af = _lstm_step(xpf, hf, cf, af, WhhAf, s < lens_a, False)
            tr = AT - 1 - s
            xpr = Ar_ref[pl.ds((CH - 1 - j) * AN, AN)]
            hr, cr, ar = _lstm_step(xpr, hr, cr, ar, WhhAr, tr < lens_a, True)
        return hf, cf, af, hr, cr, ar

    zf = jnp.zeros((FN, H), jnp.float32)
    za = jnp.zeros((AN, H), jnp.float32)
    zfh = jnp.zeros((FN, H), jnp.bfloat16)
    zah = jnp.zeros((AN, H), jnp.bfloat16)
    cf0 = (zfh, zf, zf, zfh, zf, zf)
    ca0 = (zah, za, za, zah, za, za)

    def pair_both(p, carry):
        cst, ast_ = carry
        c0 = 2 * p
        pf(c0 + 1, XPFfB_ref, XPFrB_ref)
        pa(jnp.minimum(c0 + 1, ACH - 1), XPAfB_ref, XPArB_ref)
        cst = fsteps(c0, cst, XPFfA_ref, XPFrA_ref)
        ast_ = asteps(c0, ast_, XPAfA_ref, XPArA_ref)
        pf(c0 + 2, XPFfA_ref, XPFrA_ref)
        pa(jnp.minimum(c0 + 2, ACH - 1), XPAfA_ref, XPArA_ref)
        cst = fsteps(c0 + 1, cst, XPFfB_ref, XPFrB_ref)
        ast_ = asteps(c0 + 1, ast_, XPAfB_ref, XPArB_ref)
        return cst, ast_

    def pair_facts(p, carry):
        cst, ast_ = carry
        c0 = 2 * p
        pf(c0 + 1, XPFfB_ref, XPFrB_ref)
        cst = fsteps(c0, cst, XPFfA_ref, XPFrA_ref)
        pf(jnp.minimum(c0 + 2, FCH - 1), XPFfA_ref, XPFrA_ref)
        cst = fsteps(c0 + 1, cst, XPFfB_ref, XPFrB_ref)
        return cst, ast_

    pf(0, XPFfA_ref, XPFrA_ref)
    pa(0, XPAfA_ref, XPArA_ref)
    cf1, ca1 = lax.fori_loop(0, ACH // 2, pair_both, (cf0, ca0))
    cf2, _ = lax.fori_loop(ACH // 2, FCH // 2, pair_facts, (cf1, ca1))
    enc_f = jnp.concatenate([cf2[2], cf2[5]], axis=1)
    enc_a = jnp.concatenate([ca1[2], ca1[5]], axis=1)

    iota_f = lax.broadcasted_iota(jnp.int32, (NB, FN), 1)
    iota_a = lax.broadcasted_iota(jnp.int32, (NB, AN), 1)
    fidx = fidx_ref[...]
    aidx = aidx_ref[...]
    Pf = jnp.zeros((NB, FN), jnp.float32)
    for k in range(fidx.shape[1]):
        Pf = Pf + (iota_f == fidx[:, k:k + 1]).astype(jnp.float32)
    Pa = jnp.zeros((NB, AN), jnp.float32)
    for k in range(aidx.shape[1]):
        Pa = Pa + (iota_a == aidx[:, k:k + 1]).astype(jnp.float32)
    sf = jnp.dot(Pf, enc_f, preferred_element_type=jnp.float32)
    sa = jnp.dot(Pa, enc_a, preferred_element_type=jnp.float32)

    x1 = jnp.tanh(jnp.concatenate([sf, sa], axis=1))
    inter = jnp.dot(x1, W1_ref[...], preferred_element_type=jnp.float32) + b1_ref[...]
    out_ref[...] = (jnp.dot(jnp.tanh(inter), W2_ref[...],
                            preferred_element_type=jnp.float32) + b2_ref[...])


def _tc_forward(ef_tm, ea_tm, lens_f, lens_a, fidx, aidx, *ws):
    return pl.pallas_call(
        _tc_body,
        out_shape=jax.ShapeDtypeStruct((NB, 12), jnp.float32),
        scratch_shapes=(
            [pltpu.VMEM((CH * FN, 4 * H), jnp.float32)] * 4
            + [pltpu.VMEM((CH * AN, 4 * H), jnp.float32)] * 4
        ),
        interpret=_INTERPRET,
    )(ef_tm, ea_tm, lens_f, lens_a, fidx, aidx, *ws)


def kernel(facts, fact_lens, artis, arti_lens, fact_indices, arti_indices, emb,
           fWih_f, fWhh_f, fbih_f, fbhh_f, fWih_r, fWhh_r, fbih_r, fbhh_r,
           aWih_f, aWhh_f, abih_f, abhh_f, aWih_r, aWhh_r, abih_r, abhh_r,
           W1, b1, W2, b2):
    idx_f = facts.T.reshape(-1).astype(jnp.int32)
    idx_a = artis.T.reshape(-1).astype(jnp.int32)
    ef_flat, ea_flat = _sc_gather(emb, idx_f, idx_a)
    ef_tm = ef_flat.reshape(FT, FN, D).astype(jnp.bfloat16)
    ea_tm = ea_flat.reshape(AT, AN, D).astype(jnp.bfloat16)

    bh = jnp.bfloat16
    ws = (fWih_f.T.astype(bh), fWhh_f.T.astype(bh), (fbih_f + fbhh_f)[None, :],
          fWih_r.T.astype(bh), fWhh_r.T.astype(bh), (fbih_r + fbhh_r)[None, :],
          aWih_f.T.astype(bh), aWhh_f.T.astype(bh), (abih_f + abhh_f)[None, :],
          aWih_r.T.astype(bh), aWhh_r.T.astype(bh), (abih_r + abhh_r)[None, :],
          W1.T, b1[None, :], W2.T, b2[None, :])

    return _tc_forward(
        ef_tm, ea_tm,
        fact_lens.astype(jnp.int32).reshape(FN, 1),
        arti_lens.astype(jnp.int32).reshape(AN, 1),
        fact_indices.astype(jnp.int32), arti_indices.astype(jnp.int32),
        *ws)


# speedup vs baseline: 1.4890x; 1.0237x over previous
import functools

import jax
import jax.numpy as jnp
from jax import lax
from jax.experimental import pallas as pl
from jax.experimental.pallas import tpu as pltpu
from jax.experimental.pallas import tpu_sc as plsc

FN, FT = 48, 256
AN, AT = 96, 96
D = 256
H = 256
NB = 16

_INTERPRET = False


def _sc_gather(emb, idx_f, idx_a):
    NF = FN * FT
    NA = AN * AT
    info = plsc.get_sparse_core_info()
    NC, NS = info.num_cores, info.num_subcores
    NW = NC * NS
    bf = NF // NW
    ba = NA // NW
    mesh = plsc.VectorSubcoreMesh(core_axis_name="c", subcore_axis_name="s")

    @functools.partial(
        pl.kernel, mesh=mesh,
        out_type=(jax.ShapeDtypeStruct((NF, D), jnp.float32),
                  jax.ShapeDtypeStruct((NA, D), jnp.float32)),
        scratch_types=[
            pltpu.VMEM((bf,), jnp.int32),
            pltpu.VMEM((ba,), jnp.int32),
            pltpu.VMEM((bf, D), jnp.float32),
            pltpu.SemaphoreType.DMA,
        ],
    )
    def k(emb_hbm, idxf_hbm, idxa_hbm, outf_hbm, outa_hbm,
          idxf_v, idxa_v, rows_v, sem):
        wid = lax.axis_index("s") * NC + lax.axis_index("c")
        base_f = wid * bf
        pltpu.sync_copy(idxf_hbm.at[pl.ds(base_f, bf)], idxf_v)
        pltpu.async_copy(emb_hbm.at[idxf_v], rows_v, sem).wait()
        pltpu.sync_copy(rows_v, outf_hbm.at[pl.ds(base_f, bf)])
        base_a = wid * ba
        pltpu.sync_copy(idxa_hbm.at[pl.ds(base_a, ba)], idxa_v)
        pltpu.async_copy(emb_hbm.at[idxa_v], rows_v.at[pl.ds(0, ba)], sem).wait()
        pltpu.sync_copy(rows_v.at[pl.ds(0, ba)], outa_hbm.at[pl.ds(base_a, ba)])

    return k(emb, idx_f, idx_a)


def _sigm(x):
    return jax.nn.sigmoid(x)


def _lstm_step(xp, h, c, acc, Whh, m, mask_state):
    g = xp + jnp.dot(h, Whh, preferred_element_type=jnp.float32)
    i = _sigm(g[:, 0:H])
    f = _sigm(g[:, H:2 * H])
    gg = jnp.tanh(g[:, 2 * H:3 * H])
    o = _sigm(g[:, 3 * H:4 * H])
    c_new = f * c + i * gg
    h_new = o * jnp.tanh(c_new)
    if mask_state:
        h2 = jnp.where(m, h_new, h.astype(jnp.float32)).astype(jnp.bfloat16)
        c2 = jnp.where(m, c_new, c)
    else:
        h2 = h_new.astype(jnp.bfloat16)
        c2 = c_new
    acc2 = acc + jnp.where(m, h_new, 0.0)
    return h2, c2, acc2


CH = 8
FCH = FT // CH
ACH = AT // CH


def _tc_body(ef_ref, ea_ref, lensf_ref, lensa_ref, fidx_ref, aidx_ref,
             WihFf_ref, WhhFf_ref, bFf_ref, WihFr_ref, WhhFr_ref, bFr_ref,
             WihAf_ref, WhhAf_ref, bAf_ref, WihAr_ref, WhhAr_ref, bAr_ref,
             W1_ref, b1_ref, W2_ref, b2_ref, out_ref,
             XPFfA_ref, XPFfB_ref, XPFrA_ref, XPFrB_ref,
             XPAfA_ref, XPAfB_ref, XPArA_ref, XPArB_ref):
    WhhFf = WhhFf_ref[...]
    WhhFr = WhhFr_ref[...]
    WhhAf = WhhAf_ref[...]
    WhhAr = WhhAr_ref[...]
    lens_f = lensf_ref[...]
    lens_a = lensa_ref[...]

    def pf(c, dstf, dstr):
        xf = ef_ref[pl.ds(c * CH, CH)].reshape(CH * FN, D)
        dstf[...] = (jnp.dot(xf, WihFf_ref[...],
                             preferred_element_type=jnp.float32) + bFf_ref[...])
        xr = ef_ref[pl.ds(FT - CH - c * CH, CH)].reshape(CH * FN, D)
        dstr[...] = (jnp.dot(xr, WihFr_ref[...],
                             preferred_element_type=jnp.float32) + bFr_ref[...])

    def pa(c, dstf, dstr):
        xa = ea_ref[pl.ds(c * CH, CH)].reshape(CH * AN, D)
        dstf[...] = (jnp.dot(xa, WihAf_ref[...],
                             preferred_element_type=jnp.float32) + bAf_ref[...])
        xr = ea_ref[pl.ds(AT - CH - c * CH, CH)].reshape(CH * AN, D)
        dstr[...] = (jnp.dot(xr, WihAr_ref[...],
                             preferred_element_type=jnp.float32) + bAr_ref[...])

    def fsteps(c, st, Ff_ref, Fr_ref):
        hf, cf, af, hr, cr, ar = st
        for j in range(CH):
            s = c * CH + j
            xpf = Ff_ref[pl.ds(j * FN, FN)]
            hf, cf, af = _lstm_step(xpf, hf, cf, af, WhhFf, s < lens_f, False)
            tr = FT - 1 - s
            xpr = Fr_ref[pl.ds((CH - 1 - j) * FN, FN)]
            hr, cr, ar = _lstm_step(xpr, hr, cr, ar, WhhFr, tr < lens_f, True)
        return hf, cf, af, hr, cr, ar

    def asteps(c, st, Af_ref, Ar_ref):
        hf, cf, af, hr, cr, ar = st
        for j in range(CH):
            s = c * CH + j
            xpf = Af_ref[pl.ds(j * AN, AN)]
            hf, cf, af = _lstm_step(xpf, hf, cf, af, WhhAf, s < lens_a, False)
            tr = AT - 1 - s
            xpr = Ar_ref[pl.ds((CH - 1 - j) * AN, AN)]
            hr, cr, ar = _lstm_step(xpr, hr, cr, ar, WhhAr, tr < lens_a, True)
        return hf, cf, af, hr, cr, ar

    zf = jnp.zeros((FN, H), jnp.float32)
    za = jnp.zeros((AN, H), jnp.float32)
    zfh = jnp.zeros((FN, H), jnp.bfloat16)
    zah = jnp.zeros((AN, H), jnp.bfloat16)
    cf0 = (zfh, zf, zf, zfh, zf, zf)
    ca0 = (zah, za, za, zah, za, za)

    def pair_both(p, carry):
        cst, ast_ = carry
        c0 = 2 * p
        pf(c0 + 1, XPFfB_ref, XPFrB_ref)
        pa(jnp.minimum(c0 + 1, ACH - 1), XPAfB_ref, XPArB_ref)
        cst = fsteps(c0, cst, XPFfA_ref, XPFrA_ref)
        ast_ = asteps(c0, ast_, XPAfA_ref, XPArA_ref)
        pf(c0 + 2, XPFfA_ref, XPFrA_ref)
        pa(jnp.minimum(c0 + 2, ACH - 1), XPAfA_ref, XPArA_ref)
        cst = fsteps(c0 + 1, cst, XPFfB_ref, XPFrB_ref)
        ast_ = asteps(c0 + 1, ast_, XPAfB_ref, XPArB_ref)
        return cst, ast_

    def pair_facts(p, carry):
        cst, ast_ = carry
        c0 = 2 * p
        pf(c0 + 1, XPFfB_ref, XPFrB_ref)
        cst = fsteps(c0, cst, XPFfA_ref, XPFrA_ref)
        pf(jnp.minimum(c0 + 2, FCH - 1), XPFfA_ref, XPFrA_ref)
        cst = fsteps(c0 + 1, cst, XPFfB_ref, XPFrB_ref)
        return cst, ast_

    pf(0, XPFfA_ref, XPFrA_ref)
    pa(0, XPAfA_ref, XPArA_ref)
    cf1, ca1 = lax.fori_loop(0, ACH // 2, pair_both, (cf0, ca0))
    cf2, _ = lax.fori_loop(ACH // 2, FCH // 2, pair_facts, (cf1, ca1))
    enc_f = jnp.concatenate([cf2[2], cf2[5]], axis=1)
    enc_a = jnp.concatenate([ca1[2], ca1[5]], axis=1)

    iota_f = lax.broadcasted_iota(jnp.int32, (NB, FN), 1)
    iota_a = lax.broadcasted_iota(jnp.int32, (NB, AN), 1)
    fidx = fidx_ref[...]
    aidx = aidx_ref[...]
    Pf = jnp.zeros((NB, FN), jnp.float32)
    for k in range(fidx.shape[1]):
        Pf = Pf + (iota_f == fidx[:, k:k + 1]).astype(jnp.float32)
    Pa = jnp.zeros((NB, AN), jnp.float32)
    for k in range(aidx.shape[1]):
        Pa = Pa + (iota_a == aidx[:, k:k + 1]).astype(jnp.float32)
    sf = jnp.dot(Pf, enc_f, preferred_element_type=jnp.float32)
    sa = jnp.dot(Pa, enc_a, preferred_element_type=jnp.float32)

    x1 = jnp.tanh(jnp.concatenate([sf, sa], axis=1))
    inter = jnp.dot(x1, W1_ref[...], preferred_element_type=jnp.float32) + b1_ref[...]
    out_ref[...] = (jnp.dot(jnp.tanh(inter), W2_ref[...],
                            preferred_element_type=jnp.float32) + b2_ref[...])


def _tc_forward(ef_tm, ea_tm, lens_f, lens_a, fidx, aidx, *ws):
    return pl.pallas_call(
        _tc_body,
        out_shape=jax.ShapeDtypeStruct((NB, 12), jnp.float32),
        scratch_shapes=(
            [pltpu.VMEM((CH * FN, 4 * H), jnp.float32)] * 4
            + [pltpu.VMEM((CH * AN, 4 * H), jnp.float32)] * 4
        ),
        interpret=_INTERPRET,
    )(ef_tm, ea_tm, lens_f, lens_a, fidx, aidx, *ws)


def kernel(facts, fact_lens, artis, arti_lens, fact_indices, arti_indices, emb,
           fWih_f, fWhh_f, fbih_f, fbhh_f, fWih_r, fWhh_r, fbih_r, fbhh_r,
           aWih_f, aWhh_f, abih_f, abhh_f, aWih_r, aWhh_r, abih_r, abhh_r,
           W1, b1, W2, b2):
    idx_f = facts.T.reshape(-1).astype(jnp.int32)
    idx_a = artis.T.reshape(-1).astype(jnp.int32)
    ef_flat, ea_flat = _sc_gather(emb, idx_f, idx_a)
    ef_tm = ef_flat.reshape(FT, FN, D).astype(jnp.bfloat16)
    ea_tm = ea_flat.reshape(AT, AN, D).astype(jnp.bfloat16)

    bh = jnp.bfloat16
    ws = (fWih_f.T.astype(bh), fWhh_f.T.astype(bh), (fbih_f + fbhh_f)[None, :],
          fWih_r.T.astype(bh), fWhh_r.T.astype(bh), (fbih_r + fbhh_r)[None, :],
          aWih_f.T.astype(bh), aWhh_f.T.astype(bh), (abih_f + abhh_f)[None, :],
          aWih_r.T.astype(bh), aWhh_r.T.astype(bh), (abih_r + abhh_r)[None, :],
          W1.T, b1[None, :], W2.T, b2[None, :])

    return _tc_forward(
        ef_tm, ea_tm,
        fact_lens.astype(jnp.int32).reshape(FN, 1),
        arti_lens.astype(jnp.int32).reshape(AN, 1),
        fact_indices.astype(jnp.int32), arti_indices.astype(jnp.int32),
        *ws)


# speedup vs baseline: 1.4892x; 1.0001x over previous
import functools

import jax
import jax.numpy as jnp
from jax import lax
from jax.experimental import pallas as pl
from jax.experimental.pallas import tpu as pltpu
from jax.experimental.pallas import tpu_sc as plsc

FN, FT = 48, 256
AN, AT = 96, 96
D = 256
H = 256
NB = 16


def _sc_gather(emb, idx_f, idx_a):
    NF = FN * FT
    NA = AN * AT
    info = plsc.get_sparse_core_info()
    NC, NS = info.num_cores, info.num_subcores
    NW = NC * NS
    bf = NF // NW
    ba = NA // NW
    mesh = plsc.VectorSubcoreMesh(core_axis_name="c", subcore_axis_name="s")

    @functools.partial(
        pl.kernel, mesh=mesh,
        out_type=(jax.ShapeDtypeStruct((NF, D), jnp.float32),
                  jax.ShapeDtypeStruct((NA, D), jnp.float32)),
        scratch_types=[
            pltpu.VMEM((bf,), jnp.int32),
            pltpu.VMEM((ba,), jnp.int32),
            pltpu.VMEM((bf, D), jnp.float32),
            pltpu.SemaphoreType.DMA,
        ],
    )
    def k(emb_hbm, idxf_hbm, idxa_hbm, outf_hbm, outa_hbm,
          idxf_v, idxa_v, rows_v, sem):
        wid = lax.axis_index("s") * NC + lax.axis_index("c")
        base_f = wid * bf
        pltpu.sync_copy(idxf_hbm.at[pl.ds(base_f, bf)], idxf_v)
        pltpu.async_copy(emb_hbm.at[idxf_v], rows_v, sem).wait()
        pltpu.sync_copy(rows_v, outf_hbm.at[pl.ds(base_f, bf)])
        base_a = wid * ba
        pltpu.sync_copy(idxa_hbm.at[pl.ds(base_a, ba)], idxa_v)
        pltpu.async_copy(emb_hbm.at[idxa_v], rows_v.at[pl.ds(0, ba)], sem).wait()
        pltpu.sync_copy(rows_v.at[pl.ds(0, ba)], outa_hbm.at[pl.ds(base_a, ba)])

    return k(emb, idx_f, idx_a)


def _sigm(x):
    return jax.nn.sigmoid(x)


def _lstm_step(xp, h, c, acc, Whh, m, mask_state):
    g = xp + jnp.dot(h, Whh, preferred_element_type=jnp.float32)
    i = _sigm(g[:, 0:H])
    f = _sigm(g[:, H:2 * H])
    gg = jnp.tanh(g[:, 2 * H:3 * H])
    o = _sigm(g[:, 3 * H:4 * H])
    c_new = f * c + i * gg
    h_new = o * jnp.tanh(c_new)
    if mask_state:
        h2 = jnp.where(m, h_new, h.astype(jnp.float32)).astype(jnp.bfloat16)
        c2 = jnp.where(m, c_new, c)
    else:
        h2 = h_new.astype(jnp.bfloat16)
        c2 = c_new
    acc2 = acc + jnp.where(m, h_new, 0.0)
    return h2, c2, acc2


CH = 8
FCH = FT // CH
ACH = AT // CH


def _tc_body(ef_ref, ea_ref, lensf_ref, lensa_ref, fidx_ref, aidx_ref,
             WihFf_ref, WhhFf_ref, bFf_ref, WihFr_ref, WhhFr_ref, bFr_ref,
             WihAf_ref, WhhAf_ref, bAf_ref, WihAr_ref, WhhAr_ref, bAr_ref,
             W1_ref, b1_ref, W2_ref, b2_ref, out_ref,
             XPFfA_ref, XPFfB_ref, XPFrA_ref, XPFrB_ref,
             XPAfA_ref, XPAfB_ref, XPArA_ref, XPArB_ref):
    WhhFf = WhhFf_ref[...]
    WhhFr = WhhFr_ref[...]
    WhhAf = WhhAf_ref[...]
    WhhAr = WhhAr_ref[...]
    lens_f = lensf_ref[...]
    lens_a = lensa_ref[...]

    def pf(c, dstf, dstr):
        xf = ef_ref[pl.ds(c * CH, CH)].reshape(CH * FN, D)
        dstf[...] = (jnp.dot(xf, WihFf_ref[...],
                             preferred_element_type=jnp.float32) + bFf_ref[...])
        xr = ef_ref[pl.ds(FT - CH - c * CH, CH)].reshape(CH * FN, D)
        dstr[...] = (jnp.dot(xr, WihFr_ref[...],
                             preferred_element_type=jnp.float32) + bFr_ref[...])

    def pa(c, dstf, dstr):
        xa = ea_ref[pl.ds(c * CH, CH)].reshape(CH * AN, D)
        dstf[...] = (jnp.dot(xa, WihAf_ref[...],
                             preferred_element_type=jnp.float32) + bAf_ref[...])
        xr = ea_ref[pl.ds(AT - CH - c * CH, CH)].reshape(CH * AN, D)
        dstr[...] = (jnp.dot(xr, WihAr_ref[...],
                             preferred_element_type=jnp.float32) + bAr_ref[...])

    def fsteps(c, st, Ff_ref, Fr_ref):
        hf, cf, af, hr, cr, ar = st
        for j in range(CH):
            s = c * CH + j
            xpf = Ff_ref[pl.ds(j * FN, FN)]
            hf, cf, af = _lstm_step(xpf, hf, cf, af, WhhFf, s < lens_f, False)
            tr = FT - 1 - s
            xpr = Fr_ref[pl.ds((CH - 1 - j) * FN, FN)]
            hr, cr, ar = _lstm_step(xpr, hr, cr, ar, WhhFr, tr < lens_f, True)
        return hf, cf, af, hr, cr, ar

    def asteps(c, st, Af_ref, Ar_ref):
        hf, cf, af, hr, cr, ar = st
        for j in range(CH):
            s = c * CH + j
            xpf = Af_ref[pl.ds(j * AN, AN)]
            hf, cf, af = _lstm_step(xpf, hf, cf, af, WhhAf, s < lens_a, False)
            tr = AT - 1 - s
            xpr = Ar_ref[pl.ds((CH - 1 - j) * AN, AN)]
            hr, cr, ar = _lstm_step(xpr, hr, cr, ar, WhhAr, tr < lens_a, True)
        return hf, cf, af, hr, cr, ar

    zf = jnp.zeros((FN, H), jnp.float32)
    za = jnp.zeros((AN, H), jnp.float32)
    zfh = jnp.zeros((FN, H), jnp.bfloat16)
    zah = jnp.zeros((AN, H), jnp.bfloat16)
    cf0 = (zfh, zf, zf, zfh, zf, zf)
    ca0 = (zah, za, za, zah, za, za)

    def pair_both(p, carry):
        cst, ast_ = carry
        c0 = 2 * p
        pf(c0 + 1, XPFfB_ref, XPFrB_ref)
        pa(jnp.minimum(c0 + 1, ACH - 1), XPAfB_ref, XPArB_ref)
        cst = fsteps(c0, cst, XPFfA_ref, XPFrA_ref)
        ast_ = asteps(c0, ast_, XPAfA_ref, XPArA_ref)
        pf(c0 + 2, XPFfA_ref, XPFrA_ref)
        pa(jnp.minimum(c0 + 2, ACH - 1), XPAfA_ref, XPArA_ref)
        cst = fsteps(c0 + 1, cst, XPFfB_ref, XPFrB_ref)
        ast_ = asteps(c0 + 1, ast_, XPAfB_ref, XPArB_ref)
        return cst, ast_

    def pair_facts(p, carry):
        cst, ast_ = carry
        c0 = 2 * p
        pf(c0 + 1, XPFfB_ref, XPFrB_ref)
        cst = fsteps(c0, cst, XPFfA_ref, XPFrA_ref)
        pf(jnp.minimum(c0 + 2, FCH - 1), XPFfA_ref, XPFrA_ref)
        cst = fsteps(c0 + 1, cst, XPFfB_ref, XPFrB_ref)
        return cst, ast_

    pf(0, XPFfA_ref, XPFrA_ref)
    pa(0, XPAfA_ref, XPArA_ref)
    cf1, ca1 = lax.fori_loop(0, ACH // 2, pair_both, (cf0, ca0))
    cf2, _ = lax.fori_loop(ACH // 2, FCH // 2, pair_facts, (cf1, ca1))
    enc_f = jnp.concatenate([cf2[2], cf2[5]], axis=1)
    enc_a = jnp.concatenate([ca1[2], ca1[5]], axis=1)

    iota_f = lax.broadcasted_iota(jnp.int32, (NB, FN), 1)
    iota_a = lax.broadcasted_iota(jnp.int32, (NB, AN), 1)
    fidx = fidx_ref[...]
    aidx = aidx_ref[...]
    Pf = jnp.zeros((NB, FN), jnp.float32)
    for k in range(fidx.shape[1]):
        Pf = Pf + (iota_f == fidx[:, k:k + 1]).astype(jnp.float32)
    Pa = jnp.zeros((NB, AN), jnp.float32)
    for k in range(aidx.shape[1]):
        Pa = Pa + (iota_a == aidx[:, k:k + 1]).astype(jnp.float32)
    sf = jnp.dot(Pf, enc_f, preferred_element_type=jnp.float32)
    sa = jnp.dot(Pa, enc_a, preferred_element_type=jnp.float32)

    x1 = jnp.tanh(jnp.concatenate([sf, sa], axis=1))
    inter = jnp.dot(x1, W1_ref[...], preferred_element_type=jnp.float32) + b1_ref[...]
    out_ref[...] = (jnp.dot(jnp.tanh(inter), W2_ref[...],
                            preferred_element_type=jnp.float32) + b2_ref[...])


def _tc_forward(ef_tm, ea_tm, lens_f, lens_a, fidx, aidx, *ws):
    return pl.pallas_call(
        _tc_body,
        out_shape=jax.ShapeDtypeStruct((NB, 12), jnp.float32),
        scratch_shapes=(
            [pltpu.VMEM((CH * FN, 4 * H), jnp.float32)] * 4
            + [pltpu.VMEM((CH * AN, 4 * H), jnp.float32)] * 4
        ),
    )(ef_tm, ea_tm, lens_f, lens_a, fidx, aidx, *ws)


def kernel(facts, fact_lens, artis, arti_lens, fact_indices, arti_indices, emb,
           fWih_f, fWhh_f, fbih_f, fbhh_f, fWih_r, fWhh_r, fbih_r, fbhh_r,
           aWih_f, aWhh_f, abih_f, abhh_f, aWih_r, aWhh_r, abih_r, abhh_r,
           W1, b1, W2, b2):
    idx_f = facts.T.reshape(-1).astype(jnp.int32)
    idx_a = artis.T.reshape(-1).astype(jnp.int32)
    ef_flat, ea_flat = _sc_gather(emb, idx_f, idx_a)
    ef_tm = ef_flat.reshape(FT, FN, D).astype(jnp.bfloat16)
    ea_tm = ea_flat.reshape(AT, AN, D).astype(jnp.bfloat16)

    bh = jnp.bfloat16
    ws = (fWih_f.T.astype(bh), fWhh_f.T.astype(bh), (fbih_f + fbhh_f)[None, :],
          fWih_r.T.astype(bh), fWhh_r.T.astype(bh), (fbih_r + fbhh_r)[None, :],
          aWih_f.T.astype(bh), aWhh_f.T.astype(bh), (abih_f + abhh_f)[None, :],
          aWih_r.T.astype(bh), aWhh_r.T.astype(bh), (abih_r + abhh_r)[None, :],
          W1.T, b1[None, :], W2.T, b2[None, :])

    return _tc_forward(
        ef_tm, ea_tm,
        fact_lens.astype(jnp.int32).reshape(FN, 1),
        arti_lens.astype(jnp.int32).reshape(AN, 1),
        fact_indices.astype(jnp.int32), arti_indices.astype(jnp.int32),
        *ws)
